# baseline (speedup 1.0000x reference)
import functools

import jax
import jax.numpy as jnp
from jax import lax
from jax.experimental import pallas as pl
from jax.experimental.pallas import tpu as pltpu
from jax.experimental.pallas import tpu_sc as plsc

VOCAB = 1000000
EMB_DIM = 64
HIDDEN = 128
BATCH = 16384
EPS = 1e-3

NC = 2
NS = 16
NW = NC * NS
B_TOT = 3 * BATCH
B_PER_W = B_TOT // NW
CHUNK = 128
N_CHUNKS = B_PER_W // CHUNK

_sc_mesh = plsc.VectorSubcoreMesh(core_axis_name="c", subcore_axis_name="s")


@functools.partial(
    pl.kernel,
    out_type=jax.ShapeDtypeStruct((B_TOT, 2 * EMB_DIM), jnp.float32),
    mesh=_sc_mesh,
    scratch_types=[
        pltpu.VMEM((N_CHUNKS, CHUNK), jnp.int32),
        pltpu.VMEM((CHUNK, 2 * EMB_DIM), jnp.float32),
        pltpu.SemaphoreType.DMA,
    ],
    compiler_params=pltpu.CompilerParams(
        use_tc_tiling_on_sc=True, needs_layout_passes=False
    ),
)
def _sc_gather(idx_hbm, table2_hbm, out_hbm, idx_v, rows_v, sem):
    wid = lax.axis_index("s") * NC + lax.axis_index("c")
    base = wid * B_PER_W
    pltpu.sync_copy(idx_hbm.at[wid], idx_v)

    def chunk_body(j, carry):
        pltpu.async_copy(table2_hbm.at[idx_v.at[j]], rows_v, sem).wait()
        pltpu.sync_copy(rows_v, out_hbm.at[pl.ds(base + j * CHUNK, CHUNK)])
        return carry

    lax.fori_loop(0, N_CHUNKS, chunk_body, 0, unroll=False)


BM = 2048
NB = BATCH // BM


def _mlp_body(e_ref, par_ref, w1_ref, b1_ref, gamma_ref, beta_ref, mm_ref,
              mv_ref, w2_ref, b2_ref, oa_ref, op_ref, on_ref):
    j = pl.program_id(0)
    pair = e_ref[...]
    par = par_ref[...]
    e = jnp.where(par == 0, pair[:, :EMB_DIM], pair[:, EMB_DIM:])
    h = jnp.dot(e, w1_ref[...], preferred_element_type=jnp.float32)
    h = jnp.maximum(h + b1_ref[...], 0.0)
    scale = gamma_ref[...] * lax.rsqrt(mv_ref[...] + EPS)
    shift = beta_ref[...] - mm_ref[...] * scale
    h = h * scale + shift
    o = jnp.dot(h, w2_ref[...], preferred_element_type=jnp.float32)
    o = o + b2_ref[...]

    @pl.when(j < NB)
    def _():
        oa_ref[...] = o

    @pl.when(jnp.logical_and(j >= NB, j < 2 * NB))
    def _():
        op_ref[...] = o

    @pl.when(j >= 2 * NB)
    def _():
        on_ref[...] = o


def _mlp_call(gathered, parity, w1, b1, gamma, beta, mm, mv, w2, b2):
    vec_spec = pl.BlockSpec((1, HIDDEN), lambda j: (0, 0))
    out_shape = jax.ShapeDtypeStruct((BATCH, HIDDEN), jnp.float32)

    def out_map(i):
        return lambda j, i=i: (jnp.clip(j - i * NB, 0, NB - 1), 0)

    return pl.pallas_call(
        _mlp_body,
        grid=(3 * NB,),
        in_specs=[
            pl.BlockSpec((BM, 2 * EMB_DIM), lambda j: (j, 0)),
            pl.BlockSpec((BM, 1), lambda j: (j, 0)),
            pl.BlockSpec((EMB_DIM, HIDDEN), lambda j: (0, 0)),
            vec_spec, vec_spec, vec_spec, vec_spec, vec_spec,
            pl.BlockSpec((HIDDEN, HIDDEN), lambda j: (0, 0)),
            vec_spec,
        ],
        out_specs=[
            pl.BlockSpec((BM, HIDDEN), out_map(0)),
            pl.BlockSpec((BM, HIDDEN), out_map(1)),
            pl.BlockSpec((BM, HIDDEN), out_map(2)),
        ],
        out_shape=[out_shape, out_shape, out_shape],
    )(gathered, parity, w1, b1, gamma, beta, mm, mv, w2, b2)


def kernel(anchor, positive, negative, emb_table, W1, b1, gamma, beta,
           moving_mean, moving_var, W2, b2):
    idx = jnp.concatenate([anchor, positive, negative]).astype(jnp.int32)
    pair_idx = (idx // 2).reshape(NW, N_CHUNKS, CHUNK)
    parity = (idx % 2).reshape(B_TOT, 1)

    table2 = emb_table.reshape(VOCAB // 2, 2 * EMB_DIM)
    gathered = _sc_gather(pair_idx, table2)

    b1r = b1.reshape(1, HIDDEN)
    gr = gamma.reshape(1, HIDDEN)
    br = beta.reshape(1, HIDDEN)
    mmr = moving_mean.reshape(1, HIDDEN)
    mvr = moving_var.reshape(1, HIDDEN)
    b2r = b2.reshape(1, HIDDEN)

    oa, op, on = _mlp_call(gathered, parity, W1, b1r, gr, br, mmr, mvr, W2,
                           b2r)
    return (oa, op, on)


# speedup vs baseline: 1.0508x; 1.0508x over previous
import functools

import jax
import jax.numpy as jnp
from jax import lax
from jax.experimental import pallas as pl
from jax.experimental.pallas import tpu as pltpu
from jax.experimental.pallas import tpu_sc as plsc

VOCAB = 1000000
EMB_DIM = 64
HIDDEN = 128
BATCH = 16384
EPS = 1e-3

NC = 2
NS = 16
NW = NC * NS
B_TOT = 3 * BATCH
B_PER_W = B_TOT // NW
CHUNK = 128
N_CHUNKS = B_PER_W // CHUNK

_sc_mesh = plsc.VectorSubcoreMesh(core_axis_name="c", subcore_axis_name="s")


@functools.partial(
    pl.kernel,
    out_type=jax.ShapeDtypeStruct((B_TOT, EMB_DIM), jnp.float32),
    mesh=_sc_mesh,
    scratch_types=[
        pltpu.VMEM((N_CHUNKS, CHUNK), jnp.int32),
        pltpu.VMEM((B_PER_W, EMB_DIM), jnp.float32),
        pltpu.SemaphoreType.DMA,
    ],
    compiler_params=pltpu.CompilerParams(use_tc_tiling_on_sc=False),
)
def _sc_gather(idx_hbm, table_hbm, out_hbm, idx_v, rows_v, sem):
    wid = lax.axis_index("s") * NC + lax.axis_index("c")
    base = wid * B_PER_W
    pltpu.sync_copy(idx_hbm.at[wid], idx_v)
    copies = []
    for j in range(N_CHUNKS):
        copies.append(
            pltpu.async_copy(
                table_hbm.at[idx_v.at[j]],
                rows_v.at[pl.ds(j * CHUNK, CHUNK)],
                sem,
            )
        )
    for c in copies:
        c.wait()
    pltpu.sync_copy(rows_v, out_hbm.at[pl.ds(base, B_PER_W)])


BM = 2048
NB = BATCH // BM


def _mlp_body(e_ref, w1_ref, b1_ref, gamma_ref, beta_ref, mm_ref, mv_ref,
              w2_ref, b2_ref, oa_ref, op_ref, on_ref):
    j = pl.program_id(0)
    e = e_ref[...]
    h = jnp.dot(e, w1_ref[...], preferred_element_type=jnp.float32)
    h = jnp.maximum(h + b1_ref[...], 0.0)
    scale = gamma_ref[...] * lax.rsqrt(mv_ref[...] + EPS)
    shift = beta_ref[...] - mm_ref[...] * scale
    h = h * scale + shift
    o = jnp.dot(h, w2_ref[...], preferred_element_type=jnp.float32)
    o = o + b2_ref[...]

    @pl.when(j < NB)
    def _():
        oa_ref[...] = o

    @pl.when(jnp.logical_and(j >= NB, j < 2 * NB))
    def _():
        op_ref[...] = o

    @pl.when(j >= 2 * NB)
    def _():
        on_ref[...] = o


def _mlp_call(gathered, w1, b1, gamma, beta, mm, mv, w2, b2):
    vec_spec = pl.BlockSpec((1, HIDDEN), lambda j: (0, 0))
    out_shape = jax.ShapeDtypeStruct((BATCH, HIDDEN), jnp.float32)

    def out_map(i):
        return lambda j, i=i: (jnp.clip(j - i * NB, 0, NB - 1), 0)

    return pl.pallas_call(
        _mlp_body,
        grid=(3 * NB,),
        in_specs=[
            pl.BlockSpec((BM, EMB_DIM), lambda j: (j, 0)),
            pl.BlockSpec((EMB_DIM, HIDDEN), lambda j: (0, 0)),
            vec_spec, vec_spec, vec_spec, vec_spec, vec_spec,
            pl.BlockSpec((HIDDEN, HIDDEN), lambda j: (0, 0)),
            vec_spec,
        ],
        out_specs=[
            pl.BlockSpec((BM, HIDDEN), out_map(0)),
            pl.BlockSpec((BM, HIDDEN), out_map(1)),
            pl.BlockSpec((BM, HIDDEN), out_map(2)),
        ],
        out_shape=[out_shape, out_shape, out_shape],
    )(gathered, w1, b1, gamma, beta, mm, mv, w2, b2)


def kernel(anchor, positive, negative, emb_table, W1, b1, gamma, beta,
           moving_mean, moving_var, W2, b2):
    idx = jnp.concatenate([anchor, positive, negative]).astype(jnp.int32)
    idx = idx.reshape(NW, N_CHUNKS, CHUNK)
    gathered = _sc_gather(idx, emb_table)

    b1r = b1.reshape(1, HIDDEN)
    gr = gamma.reshape(1, HIDDEN)
    br = beta.reshape(1, HIDDEN)
    mmr = moving_mean.reshape(1, HIDDEN)
    mvr = moving_var.reshape(1, HIDDEN)
    b2r = b2.reshape(1, HIDDEN)

    oa, op, on = _mlp_call(gathered, W1, b1r, gr, br, mmr, mvr, W2, b2r)
    return (oa, op, on)


# speedup vs baseline: 1.1672x; 1.1108x over previous
import functools

import jax
import jax.numpy as jnp
from jax import lax
from jax.experimental import pallas as pl
from jax.experimental.pallas import tpu as pltpu
from jax.experimental.pallas import tpu_sc as plsc

VOCAB = 1000000
EMB_DIM = 64
HIDDEN = 128
BATCH = 16384
EPS = 1e-3

NC = 2
NS = 16
NW = NC * NS
B_TOT = 3 * BATCH
B_PER_W = B_TOT // NW
CHUNK = 128
N_CHUNKS = B_PER_W // CHUNK

_sc_mesh = plsc.VectorSubcoreMesh(core_axis_name="c", subcore_axis_name="s")


@functools.partial(
    pl.kernel,
    out_type=jax.ShapeDtypeStruct((B_TOT, 2 * EMB_DIM), jnp.float32),
    mesh=_sc_mesh,
    scratch_types=[
        pltpu.VMEM((N_CHUNKS, CHUNK), jnp.int32),
        pltpu.VMEM((CHUNK, 2 * EMB_DIM), jnp.float32),
        pltpu.SemaphoreType.DMA,
    ],
    compiler_params=pltpu.CompilerParams(
        use_tc_tiling_on_sc=True, needs_layout_passes=False
    ),
)
def _sc_gather(idx_hbm, table_hbm, out_hbm, idx_v, rows_v, sem):
    wid = lax.axis_index("s") * NC + lax.axis_index("c")
    base = wid * B_PER_W
    pltpu.sync_copy(idx_hbm.at[wid], idx_v)

    def chunk_body(j, carry):
        pltpu.async_copy(table_hbm.at[idx_v.at[j]], rows_v, sem).wait()
        pltpu.sync_copy(rows_v, out_hbm.at[pl.ds(base + j * CHUNK, CHUNK)])
        return carry

    lax.fori_loop(0, N_CHUNKS, chunk_body, 0, unroll=False)


BM = 2048
NB = BATCH // BM


def _mlp_body(e_ref, w1_ref, b1_ref, gamma_ref, beta_ref, mm_ref, mv_ref,
              w2_ref, b2_ref, oa_ref, op_ref, on_ref):
    j = pl.program_id(0)
    e = e_ref[:, :EMB_DIM]
    h = jnp.dot(e, w1_ref[...], preferred_element_type=jnp.float32)
    h = jnp.maximum(h + b1_ref[...], 0.0)
    scale = gamma_ref[...] * lax.rsqrt(mv_ref[...] + EPS)
    shift = beta_ref[...] - mm_ref[...] * scale
    h = h * scale + shift
    o = jnp.dot(h, w2_ref[...], preferred_element_type=jnp.float32)
    o = o + b2_ref[...]

    @pl.when(j < NB)
    def _():
        oa_ref[...] = o

    @pl.when(jnp.logical_and(j >= NB, j < 2 * NB))
    def _():
        op_ref[...] = o

    @pl.when(j >= 2 * NB)
    def _():
        on_ref[...] = o


def _mlp_call(gathered, w1, b1, gamma, beta, mm, mv, w2, b2):
    vec_spec = pl.BlockSpec((1, HIDDEN), lambda j: (0, 0))
    out_shape = jax.ShapeDtypeStruct((BATCH, HIDDEN), jnp.float32)

    def out_map(i):
        return lambda j, i=i: (jnp.clip(j - i * NB, 0, NB - 1), 0)

    return pl.pallas_call(
        _mlp_body,
        grid=(3 * NB,),
        in_specs=[
            pl.BlockSpec((BM, 2 * EMB_DIM), lambda j: (j, 0)),
            pl.BlockSpec((EMB_DIM, HIDDEN), lambda j: (0, 0)),
            vec_spec, vec_spec, vec_spec, vec_spec, vec_spec,
            pl.BlockSpec((HIDDEN, HIDDEN), lambda j: (0, 0)),
            vec_spec,
        ],
        out_specs=[
            pl.BlockSpec((BM, HIDDEN), out_map(0)),
            pl.BlockSpec((BM, HIDDEN), out_map(1)),
            pl.BlockSpec((BM, HIDDEN), out_map(2)),
        ],
        out_shape=[out_shape, out_shape, out_shape],
    )(gathered, w1, b1, gamma, beta, mm, mv, w2, b2)


def kernel(anchor, positive, negative, emb_table, W1, b1, gamma, beta,
           moving_mean, moving_var, W2, b2):
    idx = jnp.concatenate([anchor, positive, negative]).astype(jnp.int32)
    idx = idx.reshape(NW, N_CHUNKS, CHUNK)
    padded = jnp.pad(emb_table, ((0, 0), (0, EMB_DIM)))
    gathered = _sc_gather(idx, padded)

    b1r = b1.reshape(1, HIDDEN)
    gr = gamma.reshape(1, HIDDEN)
    br = beta.reshape(1, HIDDEN)
    mmr = moving_mean.reshape(1, HIDDEN)
    mvr = moving_var.reshape(1, HIDDEN)
    b2r = b2.reshape(1, HIDDEN)

    oa, op, on = _mlp_call(gathered, W1, b1r, gr, br, mmr, mvr, W2, b2r)
    return (oa, op, on)
